# Initial kernel scaffold; baseline (speedup 1.0000x reference)
import functools

import jax
import jax.numpy as jnp
from jax import lax
from jax.experimental import pallas as pl
from jax.experimental.pallas import tpu as pltpu
from jax.experimental.pallas import tpu_sc as plsc

N_ROWS = 320000
N_SEG = 10000
D = 128
NC = 2
NS = 16
NW = NC * NS
SEG_PER_W = 320
SEG_PAD = NW * SEG_PER_W
PTR_SLICE = SEG_PER_W + 8
PTR_PAD = (NW - 1) * SEG_PER_W + PTR_SLICE
CHUNK = 512
LANES = 16
KD = D // LANES


def _sc_body(src_hbm, ptr_hbm, out_hbm, ptr_v, buf, acc, sem):
    wid = lax.axis_index("s") * NC + lax.axis_index("c")
    seg0 = wid * SEG_PER_W

    pltpu.sync_copy(ptr_hbm.at[pl.ds(seg0, PTR_SLICE)], ptr_v)
    row_lo = ptr_v[0]
    row_hi = ptr_v[SEG_PER_W]

    zeros = jnp.zeros((LANES,), jnp.float32)

    def zero_body(s, _):
        for k in range(KD):
            acc[s, pl.ds(k * LANES, LANES)] = zeros
        return 0

    lax.fori_loop(0, SEG_PER_W, zero_body, 0)

    nrows = row_hi - row_lo
    nch = (nrows + CHUNK - 1) // CHUNK

    def chunk_body(c, cur):
        off = row_lo + c * CHUNK
        off_c = jnp.minimum(off, N_ROWS - CHUNK)
        pltpu.sync_copy(src_hbm.at[pl.ds(off_c, CHUNK)], buf)

        def row_body(r, cur):
            idx = off_c + r
            valid = (idx >= off) & (idx < row_hi)

            def adv_cond(cu):
                return valid & (ptr_v[cu + 1] <= idx)

            cur = lax.while_loop(adv_cond, lambda cu: cu + 1, cur)

            @pl.when(valid)
            def _():
                for k in range(KD):
                    v = buf[r, pl.ds(k * LANES, LANES)]
                    plsc.addupdate(acc.at[cur, pl.ds(k * LANES, LANES)], v)

            return cur

        return lax.fori_loop(0, CHUNK, row_body, cur)

    lax.fori_loop(0, nch, chunk_body, 0)

    def div_body(s, _):
        cnt = ptr_v[s + 1] - ptr_v[s]
        recip = 1.0 / jnp.maximum(cnt.astype(jnp.float32), 1.0)
        rv = jnp.full((LANES,), recip, jnp.float32)
        for k in range(KD):
            sl = pl.ds(k * LANES, LANES)
            acc[s, sl] = acc[s, sl] * rv
        return 0

    lax.fori_loop(0, SEG_PER_W, div_body, 0)

    pltpu.sync_copy(acc, out_hbm.at[pl.ds(seg0, SEG_PER_W)])


@jax.jit
def _run(src, ptr_pad):
    mesh = plsc.VectorSubcoreMesh(core_axis_name="c", subcore_axis_name="s")
    k = pl.kernel(
        _sc_body,
        out_type=jax.ShapeDtypeStruct((SEG_PAD, D), jnp.float32),
        mesh=mesh,
        scratch_types=[
            pltpu.VMEM((PTR_SLICE,), jnp.int32),
            pltpu.VMEM((CHUNK, D), jnp.float32),
            pltpu.VMEM((SEG_PER_W, D), jnp.float32),
            pltpu.SemaphoreType.DMA,
        ],
    )
    return k(src, ptr_pad)


def kernel(src, indptr):
    ptr = indptr.astype(jnp.int32)
    ptr_pad = jnp.concatenate(
        [ptr, jnp.full((PTR_PAD - ptr.shape[0],), ptr[-1], jnp.int32)]
    )
    out = _run(src, ptr_pad)
    return out[:N_SEG]


# speedup vs baseline: 128.4568x; 128.4568x over previous
import functools

import jax
import jax.numpy as jnp
from jax import lax
from jax.experimental import pallas as pl
from jax.experimental.pallas import tpu as pltpu
from jax.experimental.pallas import tpu_sc as plsc

N_ROWS = 320000
N_SEG = 10000
D = 128
NC = 2
NS = 16
NW = NC * NS
SEG_PER_W = 320
SEG_PAD = NW * SEG_PER_W
PTR_SLICE = SEG_PER_W + 24
PTR_PAD = (NW - 1) * SEG_PER_W + PTR_SLICE
CHUNK = 512
LANES = 16
KD = D // LANES
NCH = CHUNK // 128
G = SEG_PER_W // LANES
ACC_ROWS = NS * SEG_PER_W + NS


def _pread(ref, i):
    return ref[pl.ds(i, LANES)][0]


def _sc_body(src_hbm, ptr_hbm, out_hbm, ptr_v, buf, marks, ids0, ids1, ids2, ids3, acc, sem):
    sid = lax.axis_index("s")
    cid = lax.axis_index("c")
    wid = sid * NC + cid
    seg0 = pl.multiple_of(wid * SEG_PER_W, 8)
    slot0 = pl.multiple_of(sid * SEG_PER_W, 8)
    dummy = NS * SEG_PER_W + sid

    ids_refs = (ids0, ids1, ids2, ids3)
    pltpu.sync_copy(ptr_hbm.at[pl.ds(seg0, PTR_SLICE)], ptr_v)
    row_lo = _pread(ptr_v, 0)
    row_hi = _pread(ptr_v, SEG_PER_W)

    zf = jnp.zeros((LANES,), jnp.float32)
    zi = jnp.zeros((LANES,), jnp.int32)
    ones = jnp.ones((LANES,), jnp.int32)
    iota = lax.iota(jnp.int32, LANES)

    def zero_body(s, _):
        for k in range(KD):
            buf[s, pl.ds(k * LANES, LANES)] = zf
        return 0

    lax.fori_loop(0, SEG_PER_W, zero_body, 0)
    pltpu.sync_copy(buf.at[pl.ds(0, SEG_PER_W)], acc.at[pl.ds(slot0, SEG_PER_W)])

    row_lo_a = (row_lo // 8) * 8
    nch = (row_hi - row_lo_a + CHUNK - 1) // CHUNK

    def chunk_body(c, base):
        off = row_lo_a + c * CHUNK
        off_c = pl.multiple_of(jnp.minimum(off, N_ROWS - CHUNK), 8)
        pltpu.sync_copy(src_hbm.at[pl.ds(off_c, CHUNK)], buf)

        def zm(j, _):
            marks[pl.ds(j * LANES, LANES)] = zi
            return 0

        lax.fori_loop(0, CHUNK // LANES, zm, 0)

        hi = off_c + CHUNK

        def sm(g, _):
            starts = ptr_v[pl.ds(g * LANES, LANES)]
            m = (starts >= off) & (starts < hi)
            plsc.addupdate_scatter(marks, [starts - off_c], ones, mask=m)
            return 0

        lax.fori_loop(0, G, sm, 0)

        for q in range(NCH):
            for jj in range(8):
                mk = marks[pl.ds(q * 128 + jj * LANES, LANES)]
                csum = plsc.cumsum(mk)
                idx16 = off_c + q * 128 + jj * LANES + iota
                valid = (idx16 >= off) & (idx16 >= row_lo) & (idx16 < row_hi)
                slot = jnp.where(valid, slot0 + base + csum - 1, dummy)
                ids_refs[q][pl.ds(jj * LANES, LANES)] = slot
                base = base + csum[15]

        for q in range(NCH):
            pltpu.sync_copy(
                buf.at[pl.ds(q * 128, 128)], acc.at[ids_refs[q]], add=True
            )
        return base

    lax.fori_loop(0, nch, chunk_body, 0)

    pltpu.sync_copy(acc.at[pl.ds(slot0, SEG_PER_W)], buf.at[pl.ds(0, SEG_PER_W)])

    def div_body(g, _):
        cur16 = ptr_v[pl.ds(g * LANES, LANES)]
        nxt16 = plsc.load_gather(ptr_v, [g * LANES + 1 + iota])
        cntf = (nxt16 - cur16).astype(jnp.float32)
        recip = 1.0 / jnp.maximum(cntf, 1.0)
        for jj in range(LANES):
            rv = jnp.full((LANES,), recip[jj], jnp.float32)
            for k in range(KD):
                sl = pl.ds(k * LANES, LANES)
                buf[g * LANES + jj, sl] = buf[g * LANES + jj, sl] * rv
        return 0

    lax.fori_loop(0, G, div_body, 0)

    pltpu.sync_copy(buf.at[pl.ds(0, SEG_PER_W)], out_hbm.at[pl.ds(seg0, SEG_PER_W)])


@jax.jit
def _run(src, ptr_pad):
    mesh = plsc.VectorSubcoreMesh(core_axis_name="c", subcore_axis_name="s")
    k = pl.kernel(
        _sc_body,
        out_type=jax.ShapeDtypeStruct((SEG_PAD, D), jnp.float32),
        mesh=mesh,
        scratch_types=[
            pltpu.VMEM((PTR_SLICE,), jnp.int32),
            pltpu.VMEM((CHUNK, D), jnp.float32),
            pltpu.VMEM((CHUNK,), jnp.int32),
            pltpu.VMEM((128,), jnp.int32),
            pltpu.VMEM((128,), jnp.int32),
            pltpu.VMEM((128,), jnp.int32),
            pltpu.VMEM((128,), jnp.int32),
            pltpu.VMEM_SHARED((ACC_ROWS, D), jnp.float32),
            pltpu.SemaphoreType.DMA,
        ],
        compiler_params=pltpu.CompilerParams(needs_layout_passes=False),
    )
    return k(src, ptr_pad)


def kernel(src, indptr):
    ptr = indptr.astype(jnp.int32)
    ptr_pad = jnp.concatenate(
        [ptr, jnp.full((PTR_PAD - ptr.shape[0],), ptr[-1], jnp.int32)]
    )
    out = _run(src, ptr_pad)
    return out[:N_SEG]


# speedup vs baseline: 171.8217x; 1.3376x over previous
import jax
import jax.numpy as jnp
from jax import lax
from jax.experimental import pallas as pl
from jax.experimental.pallas import tpu as pltpu
from jax.experimental.pallas import tpu_sc as plsc

N_ROWS = 320000
N_SEG = 10000
D = 128
NC = 2
NS = 16
NW = NC * NS
SEG_PER_W = 320
SEG_PAD = NW * SEG_PER_W
PTR_SLICE = SEG_PER_W + 24
PTR_PAD = (NW - 1) * SEG_PER_W + PTR_SLICE
CHUNK = 256
LANES = 16
KD = D // LANES
NCH = CHUNK // 128
G = SEG_PER_W // LANES
ACC_ROWS = NS * SEG_PER_W + NS


def _pread(ref, i):
    return ref[pl.ds(i, LANES)][0]


def _sc_body(src_hbm, ptr_hbm, out_hbm, ptr_v, buf_a, buf_b, marks,
             ids0, ids1, acc, sem_a, sem_b, sem_sc):
    sid = lax.axis_index("s")
    cid = lax.axis_index("c")
    wid = sid * NC + cid
    seg0 = pl.multiple_of(wid * SEG_PER_W, 8)
    slot0 = pl.multiple_of(sid * SEG_PER_W, 8)
    dummy = NS * SEG_PER_W + sid

    ids_refs = (ids0, ids1)
    pltpu.sync_copy(ptr_hbm.at[pl.ds(seg0, PTR_SLICE)], ptr_v)
    row_lo = _pread(ptr_v, 0)
    row_hi = _pread(ptr_v, SEG_PER_W)

    zf = jnp.zeros((LANES,), jnp.float32)
    zi = jnp.zeros((LANES,), jnp.int32)
    ones = jnp.ones((LANES,), jnp.int32)
    iota = lax.iota(jnp.int32, LANES)

    def zero_body(s, _):
        for k in range(KD):
            buf_a[s, pl.ds(k * LANES, LANES)] = zf
        return 0

    lax.fori_loop(0, SEG_PER_W, zero_body, 0)
    pltpu.sync_copy(buf_a.at[pl.ds(0, SEG_PER_W)], acc.at[pl.ds(slot0, SEG_PER_W)])

    row_lo_a = (row_lo // 8) * 8
    nch = (row_hi - row_lo_a + CHUNK - 1) // CHUNK

    def chunk_off(c):
        off = row_lo_a + c * CHUNK
        return off, pl.multiple_of(jnp.minimum(off, N_ROWS - CHUNK), 8)

    def start_load(c, buf, sem):
        _, off_c = chunk_off(c)
        pltpu.async_copy(src_hbm.at[pl.ds(off_c, CHUNK)], buf, sem)

    @pl.when(0 < nch)
    def _():
        start_load(0, buf_a, sem_a)

    @pl.when(1 < nch)
    def _():
        start_load(1, buf_b, sem_b)

    def do_chunk(c, buf, sem, base):
        off, off_c = chunk_off(c)

        def zm(j, _):
            marks[pl.ds(j * LANES, LANES)] = zi
            return 0

        lax.fori_loop(0, CHUNK // LANES, zm, 0)

        hi = off_c + CHUNK

        def sm(g, _):
            starts = ptr_v[pl.ds(g * LANES, LANES)]
            m = (starts >= off) & (starts < hi)
            plsc.addupdate_scatter(marks, [starts - off_c], ones, mask=m)
            return 0

        lax.fori_loop(0, G, sm, 0)

        for q in range(NCH):
            for jj in range(8):
                mk = marks[pl.ds(q * 128 + jj * LANES, LANES)]
                csum = plsc.cumsum(mk)
                idx16 = off_c + q * 128 + jj * LANES + iota
                valid = (idx16 >= off) & (idx16 >= row_lo) & (idx16 < row_hi)
                slot = jnp.where(valid, slot0 + base + csum - 1, dummy)
                ids_refs[q][pl.ds(jj * LANES, LANES)] = slot
                base = base + csum[15]

        pltpu.make_async_copy(src_hbm.at[pl.ds(off_c, CHUNK)], buf, sem).wait()
        descs = [
            pltpu.async_copy(
                buf.at[pl.ds(q * 128, 128)], acc.at[ids_refs[q]], sem_sc, add=True
            )
            for q in range(NCH)
        ]
        for d in descs:
            d.wait()

        @pl.when(c + 2 < nch)
        def _():
            start_load(c + 2, buf, sem)

        return base

    def pair_body(g, base):
        c0 = g * 2
        base = lax.cond(
            c0 < nch, lambda bs: do_chunk(c0, buf_a, sem_a, bs),
            lambda bs: bs, base,
        )
        c1 = g * 2 + 1
        base = lax.cond(
            c1 < nch, lambda bs: do_chunk(c1, buf_b, sem_b, bs),
            lambda bs: bs, base,
        )
        return base

    lax.fori_loop(0, (nch + 1) // 2, pair_body, 0)

    pltpu.sync_copy(acc.at[pl.ds(slot0, SEG_PER_W)], buf_a.at[pl.ds(0, SEG_PER_W)])

    def div_body(g, _):
        cur16 = ptr_v[pl.ds(g * LANES, LANES)]
        nxt16 = plsc.load_gather(ptr_v, [g * LANES + 1 + iota])
        cntf = (nxt16 - cur16).astype(jnp.float32)
        recip = 1.0 / jnp.maximum(cntf, 1.0)
        for jj in range(LANES):
            rv = jnp.full((LANES,), recip[jj], jnp.float32)
            for k in range(KD):
                sl = pl.ds(k * LANES, LANES)
                buf_a[g * LANES + jj, sl] = buf_a[g * LANES + jj, sl] * rv
        return 0

    lax.fori_loop(0, G, div_body, 0)

    pltpu.sync_copy(buf_a.at[pl.ds(0, SEG_PER_W)], out_hbm.at[pl.ds(seg0, SEG_PER_W)])


@jax.jit
def _run(src, ptr_pad):
    mesh = plsc.VectorSubcoreMesh(core_axis_name="c", subcore_axis_name="s")
    k = pl.kernel(
        _sc_body,
        out_type=jax.ShapeDtypeStruct((SEG_PAD, D), jnp.float32),
        mesh=mesh,
        scratch_types=[
            pltpu.VMEM((PTR_SLICE,), jnp.int32),
            pltpu.VMEM((CHUNK, D), jnp.float32),
            pltpu.VMEM((CHUNK, D), jnp.float32),
            pltpu.VMEM((CHUNK,), jnp.int32),
            pltpu.VMEM((128,), jnp.int32),
            pltpu.VMEM((128,), jnp.int32),
            pltpu.VMEM_SHARED((ACC_ROWS, D), jnp.float32),
            pltpu.SemaphoreType.DMA,
            pltpu.SemaphoreType.DMA,
            pltpu.SemaphoreType.DMA,
        ],
        compiler_params=pltpu.CompilerParams(needs_layout_passes=False),
    )
    return k(src, ptr_pad)


def kernel(src, indptr):
    ptr = indptr.astype(jnp.int32)
    ptr_pad = jnp.concatenate(
        [ptr, jnp.full((PTR_PAD - ptr.shape[0],), ptr[-1], jnp.int32)]
    )
    out = _run(src, ptr_pad)
    return out[:N_SEG]


# speedup vs baseline: 184.1932x; 1.0720x over previous
import jax
import jax.numpy as jnp
from jax import lax
from jax.experimental import pallas as pl
from jax.experimental.pallas import tpu as pltpu
from jax.experimental.pallas import tpu_sc as plsc

N_ROWS = 320000
N_SEG = 10000
D = 128
NC = 2
NS = 16
NW = NC * NS
SEG_PER_W = 320
SEG_PAD = NW * SEG_PER_W
PTR_SLICE = SEG_PER_W + 24
PTR_PAD = (NW - 1) * SEG_PER_W + PTR_SLICE
GROUP = 128
NBUF = 4
LANES = 16
KD = D // LANES
G = SEG_PER_W // LANES
ACC_ROWS = NS * SEG_PER_W + NS


def _pread(ref, i):
    return ref[pl.ds(i, LANES)][0]


def _sc_body(src_hbm, ptr_hbm, out_hbm, ptr_v, marks,
             buf0, buf1, buf2, buf3, ids0, ids1, ids2, ids3, acc,
             sem0, sem1, sem2, sem3, sem_sc):
    sid = lax.axis_index("s")
    cid = lax.axis_index("c")
    wid = sid * NC + cid
    seg0 = pl.multiple_of(wid * SEG_PER_W, 8)
    slot0 = pl.multiple_of(sid * SEG_PER_W, 8)
    dummy = NS * SEG_PER_W + sid

    bufs = (buf0, buf1, buf2, buf3)
    ids_refs = (ids0, ids1, ids2, ids3)
    sems = (sem0, sem1, sem2, sem3)

    pltpu.sync_copy(ptr_hbm.at[pl.ds(seg0, PTR_SLICE)], ptr_v)
    row_lo = _pread(ptr_v, 0)
    row_hi = _pread(ptr_v, SEG_PER_W)

    zf = jnp.zeros((LANES,), jnp.float32)
    zi = jnp.zeros((LANES,), jnp.int32)
    ones = jnp.ones((LANES,), jnp.int32)
    iota = lax.iota(jnp.int32, LANES)

    def zero_body(s, _):
        for k in range(KD):
            buf0[s, pl.ds(k * LANES, LANES)] = zf
        return 0

    lax.fori_loop(0, GROUP, zero_body, 0)
    for p, m in ((0, 128), (128, 128), (256, 64)):
        pltpu.sync_copy(buf0.at[pl.ds(0, m)], acc.at[pl.ds(slot0 + p, m)])

    row_lo_a = (row_lo // 8) * 8
    ngrp = (row_hi - row_lo_a + GROUP - 1) // GROUP

    def grp_off(t):
        off = row_lo_a + t * GROUP
        return off, pl.multiple_of(jnp.minimum(off, N_ROWS - GROUP), 8)

    def start_load(t, buf, sem):
        _, off_c = grp_off(t)
        pltpu.async_copy(src_hbm.at[pl.ds(off_c, GROUP)], buf, sem)

    @pl.when(0 < ngrp)
    def _():
        start_load(0, buf0, sem0)

    @pl.when(1 < ngrp)
    def _():
        start_load(1, buf1, sem1)

    def ring_body(g, base):
        for k in range(NBUF):
            t = g * NBUF + k
            kk = (k + 2) % NBUF

            @pl.when((t >= 2) & (t - 2 < ngrp))
            def _(kk=kk):
                pltpu.make_async_copy(
                    bufs[kk], acc.at[ids_refs[kk]], sem_sc
                ).wait()

            @pl.when(t + 2 < ngrp)
            def _(kk=kk, t=t):
                start_load(t + 2, bufs[kk], sems[kk])

            def fire(bs, t=t, k=k):
                off, off_c = grp_off(t)
                for j in range(GROUP // LANES):
                    marks[pl.ds(j * LANES, LANES)] = zi
                hi = off_c + GROUP

                def sm(q, _):
                    starts = ptr_v[pl.ds(q * LANES, LANES)]
                    m = (starts >= off) & (starts < hi)
                    plsc.addupdate_scatter(marks, [starts - off_c], ones, mask=m)
                    return 0

                lax.fori_loop(0, G, sm, 0)

                for j in range(GROUP // LANES):
                    mk = marks[pl.ds(j * LANES, LANES)]
                    csum = plsc.cumsum(mk)
                    idx16 = off_c + j * LANES + iota
                    valid = (idx16 >= off) & (idx16 >= row_lo) & (idx16 < row_hi)
                    slot = jnp.where(valid, slot0 + bs + csum - 1, dummy)
                    ids_refs[k][pl.ds(j * LANES, LANES)] = slot
                    bs = bs + csum[15]

                pltpu.make_async_copy(
                    src_hbm.at[pl.ds(off_c, GROUP)], bufs[k], sems[k]
                ).wait()
                pltpu.async_copy(
                    bufs[k], acc.at[ids_refs[k]], sem_sc, add=True
                )
                return bs

            base = lax.cond(t < ngrp, fire, lambda bs: bs, base)
        return base

    lax.fori_loop(0, (ngrp + 2 + NBUF - 1) // NBUF, ring_body, 0)

    for p, m in ((0, 128), (128, 128), (256, 64)):
        pltpu.sync_copy(acc.at[pl.ds(slot0 + p, m)], buf0.at[pl.ds(0, m)])

        def div_body(g2, _, p=p):
            cur16 = ptr_v[pl.ds(p + g2 * LANES, LANES)]
            nxt16 = plsc.load_gather(ptr_v, [p + g2 * LANES + 1 + iota])
            cntf = (nxt16 - cur16).astype(jnp.float32)
            recip = 1.0 / jnp.maximum(cntf, 1.0)
            for jj in range(LANES):
                rv = jnp.full((LANES,), recip[jj], jnp.float32)
                for k in range(KD):
                    sl = pl.ds(k * LANES, LANES)
                    buf0[g2 * LANES + jj, sl] = buf0[g2 * LANES + jj, sl] * rv
            return 0

        lax.fori_loop(0, m // LANES, div_body, 0)
        pltpu.sync_copy(buf0.at[pl.ds(0, m)], out_hbm.at[pl.ds(seg0 + p, m)])


@jax.jit
def _run(src, ptr_pad):
    mesh = plsc.VectorSubcoreMesh(core_axis_name="c", subcore_axis_name="s")
    k = pl.kernel(
        _sc_body,
        out_type=jax.ShapeDtypeStruct((SEG_PAD, D), jnp.float32),
        mesh=mesh,
        scratch_types=[
            pltpu.VMEM((PTR_SLICE,), jnp.int32),
            pltpu.VMEM((GROUP,), jnp.int32),
            pltpu.VMEM((GROUP, D), jnp.float32),
            pltpu.VMEM((GROUP, D), jnp.float32),
            pltpu.VMEM((GROUP, D), jnp.float32),
            pltpu.VMEM((GROUP, D), jnp.float32),
            pltpu.VMEM((GROUP,), jnp.int32),
            pltpu.VMEM((GROUP,), jnp.int32),
            pltpu.VMEM((GROUP,), jnp.int32),
            pltpu.VMEM((GROUP,), jnp.int32),
            pltpu.VMEM_SHARED((ACC_ROWS, D), jnp.float32),
            pltpu.SemaphoreType.DMA,
            pltpu.SemaphoreType.DMA,
            pltpu.SemaphoreType.DMA,
            pltpu.SemaphoreType.DMA,
            pltpu.SemaphoreType.DMA,
        ],
        compiler_params=pltpu.CompilerParams(needs_layout_passes=False),
    )
    return k(src, ptr_pad)


def kernel(src, indptr):
    ptr = indptr.astype(jnp.int32)
    ptr_pad = jnp.concatenate(
        [ptr, jnp.full((PTR_PAD - ptr.shape[0],), ptr[-1], jnp.int32)]
    )
    out = _run(src, ptr_pad)
    return out[:N_SEG]
